# Initial kernel scaffold; baseline (speedup 1.0000x reference)
import jax
import jax.numpy as jnp
from jax.experimental import pallas as pl


def kernel(userId, itemId, neg_itemId, user_emb, item_emb, edge_row, edge_col, edge_weight):
    raise NotImplementedError("write your pallas kernel here")


# speedup vs baseline: 3.5074x; 3.5074x over previous
import functools

import jax
import jax.numpy as jnp
from jax import lax
from jax.experimental import pallas as pl
from jax.experimental.pallas import tpu as pltpu
from jax.experimental.pallas import tpu_sc as plsc

N_U = 50000
N_I = 50000
N = N_U + N_I
D = 64
CHUNK = 25000
ACC_ROWS = CHUNK + 8
EB = 128
ROWS_PT = 1560


def _spmm_body(Upad, in_hbm, row_hbm, col_hbm, w_hbm, zeros_hbm, out_hbm,
               acc, rows_v, col_v, row_v, w_v, idx_v, zbuf, sem):
    c = lax.axis_index("c")
    s = lax.axis_index("s")
    per_tile = Upad // 16
    nb = per_tile // EB

    pltpu.sync_copy(zeros_hbm, zbuf)

    for k in range(2):
        chunk_lo = c * N_U + k * CHUNK

        zstart = s * ROWS_PT
        for j in range(ROWS_PT // EB):
            pltpu.sync_copy(zbuf, acc.at[pl.ds(zstart + j * EB, EB)])
        rem = ROWS_PT - (ROWS_PT // EB) * EB
        if rem:
            pltpu.sync_copy(zbuf.at[pl.ds(0, rem)],
                            acc.at[pl.ds(zstart + (ROWS_PT // EB) * EB, rem)])

        @pl.when(s == 0)
        def _():
            pltpu.sync_copy(zbuf.at[pl.ds(0, 48)],
                            acc.at[pl.ds(16 * ROWS_PT, 48)])

        plsc.subcore_barrier()

        ebase0 = c * Upad + s * per_tile

        def batch_body(b, carry):
            ebase = ebase0 + b * EB
            pltpu.sync_copy(col_hbm.at[pl.ds(ebase, EB)], col_v)
            pltpu.sync_copy(row_hbm.at[pl.ds(ebase, EB)], row_v)
            pltpu.sync_copy(w_hbm.at[pl.ds(ebase, EB)], w_v)
            pltpu.async_copy(in_hbm.at[col_v], rows_v, sem).wait()

            for i in range(EB // 16):
                rv = row_v[pl.ds(i * 16, 16)]
                local = rv - chunk_lo
                ok = (local >= 0) & (local < CHUNK)
                idx_v[pl.ds(i * 16, 16)] = jnp.where(ok, local, CHUNK)

            for i in range(EB // 16):
                wv = w_v[pl.ds(i * 16, 16)]
                for j in range(16):
                    e = i * 16 + j
                    ws = wv[j]
                    for q in range(D // 16):
                        rq = rows_v[e, pl.ds(q * 16, 16)]
                        rows_v[e, pl.ds(q * 16, 16)] = rq * ws

            pltpu.sync_copy(rows_v, acc.at[idx_v], add=True)
            return carry

        lax.fori_loop(0, nb, batch_body, 0)

        plsc.subcore_barrier()

        for j in range(ROWS_PT // EB):
            pltpu.sync_copy(acc.at[pl.ds(zstart + j * EB, EB)],
                            out_hbm.at[pl.ds(chunk_lo + zstart + j * EB, EB)])
        if rem:
            pltpu.sync_copy(
                acc.at[pl.ds(zstart + (ROWS_PT // EB) * EB, rem)],
                out_hbm.at[pl.ds(chunk_lo + zstart + (ROWS_PT // EB) * EB, rem)])

        @pl.when(s == 0)
        def _():
            pltpu.sync_copy(acc.at[pl.ds(16 * ROWS_PT, 40)],
                            out_hbm.at[pl.ds(chunk_lo + 16 * ROWS_PT, 40)])

        plsc.subcore_barrier()


def _make_spmm(Upad):
    mesh = plsc.VectorSubcoreMesh(core_axis_name="c", subcore_axis_name="s")
    return functools.partial(
        pl.kernel,
        functools.partial(_spmm_body, Upad),
        mesh=mesh,
        compiler_params=pltpu.CompilerParams(use_tc_tiling_on_sc=False),
        out_type=jax.ShapeDtypeStruct((N, D), jnp.float32),
        scratch_types=[
            pltpu.VMEM_SHARED((ACC_ROWS, D), jnp.float32),
            pltpu.VMEM((EB, D), jnp.float32),
            pltpu.VMEM((EB,), jnp.int32),
            pltpu.VMEM((EB,), jnp.int32),
            pltpu.VMEM((EB,), jnp.float32),
            pltpu.VMEM((EB,), jnp.int32),
            pltpu.VMEM((EB, D), jnp.float32),
            pltpu.SemaphoreType.DMA,
        ],
    )()


BPT = 128


def _final_body(e0, e1, e2, e3, ue_hbm, ie_hbm, uid_hbm, pid_hbm, nid_hbm,
                pos_hbm, neg_hbm, uego_hbm, pego_hbm, nego_hbm,
                uid_v, pid_v, nid_v, sid_v, b0, b1, b2, b3, fu, fp, fn,
                pos_v, neg_v, sem):
    c = lax.axis_index("c")
    s = lax.axis_index("s")
    wid = s * 2 + c
    base = wid * BPT

    pltpu.sync_copy(uid_hbm.at[pl.ds(base, BPT)], uid_v)
    pltpu.sync_copy(pid_hbm.at[pl.ds(base, BPT)], pid_v)
    pltpu.sync_copy(nid_hbm.at[pl.ds(base, BPT)], nid_v)

    pltpu.async_copy(ue_hbm.at[uid_v], b0, sem).wait()
    pltpu.sync_copy(b0, uego_hbm.at[pl.ds(base, BPT)])
    pltpu.async_copy(ie_hbm.at[pid_v], b0, sem).wait()
    pltpu.sync_copy(b0, pego_hbm.at[pl.ds(base, BPT)])
    pltpu.async_copy(ie_hbm.at[nid_v], b0, sem).wait()
    pltpu.sync_copy(b0, nego_hbm.at[pl.ds(base, BPT)])

    def combine(idv, dst):
        pltpu.async_copy(e0.at[idv], b0, sem).wait()
        pltpu.async_copy(e1.at[idv], b1, sem).wait()
        pltpu.async_copy(e2.at[idv], b2, sem).wait()
        pltpu.async_copy(e3.at[idv], b3, sem).wait()

        def row_body(r, carry):
            for q in range(D // 16):
                sl = pl.ds(q * 16, 16)
                v = (b0[r, sl] + b1[r, sl] + b2[r, sl] + b3[r, sl]) * 0.25
                dst[r, sl] = v
            return carry

        lax.fori_loop(0, BPT, row_body, 0)

    combine(uid_v, fu)

    for i in range(BPT // 16):
        sl = pl.ds(i * 16, 16)
        sid_v[sl] = pid_v[sl] + N_U
    combine(sid_v, fp)
    for i in range(BPT // 16):
        sl = pl.ds(i * 16, 16)
        sid_v[sl] = nid_v[sl] + N_U
    combine(sid_v, fn)

    lane = jax.lax.iota(jnp.int32, 16)
    for g in range(BPT // 16):
        def dot_body(j, carry):
            vp, vn = carry
            e = g * 16 + j
            accp = jnp.zeros((16,), jnp.float32)
            accn = jnp.zeros((16,), jnp.float32)
            for q in range(D // 16):
                sl = pl.ds(q * 16, 16)
                uvq = fu[e, sl]
                accp = accp + uvq * fp[e, sl]
                accn = accn + uvq * fn[e, sl]
            sp = accp[0]
            sn = accn[0]
            for t in range(1, 16):
                sp = sp + accp[t]
                sn = sn + accn[t]
            vp = jnp.where(lane == j, sp, vp)
            vn = jnp.where(lane == j, sn, vn)
            return vp, vn

        zp = jnp.zeros((16,), jnp.float32)
        vp, vn = lax.fori_loop(0, 16, dot_body, (zp, zp))
        pos_v[pl.ds(g * 16, 16)] = vp
        neg_v[pl.ds(g * 16, 16)] = vn

    pltpu.sync_copy(pos_v, pos_hbm.at[pl.ds(base, BPT)])
    pltpu.sync_copy(neg_v, neg_hbm.at[pl.ds(base, BPT)])


def _make_final():
    mesh = plsc.VectorSubcoreMesh(core_axis_name="c", subcore_axis_name="s")
    B = 4096
    return functools.partial(
        pl.kernel,
        _final_body,
        mesh=mesh,
        compiler_params=pltpu.CompilerParams(use_tc_tiling_on_sc=False),
        out_type=(
            jax.ShapeDtypeStruct((B,), jnp.float32),
            jax.ShapeDtypeStruct((B,), jnp.float32),
            jax.ShapeDtypeStruct((B, D), jnp.float32),
            jax.ShapeDtypeStruct((B, D), jnp.float32),
            jax.ShapeDtypeStruct((B, D), jnp.float32),
        ),
        scratch_types=[
            pltpu.VMEM((BPT,), jnp.int32),
            pltpu.VMEM((BPT,), jnp.int32),
            pltpu.VMEM((BPT,), jnp.int32),
            pltpu.VMEM((BPT,), jnp.int32),
            pltpu.VMEM((BPT, D), jnp.float32),
            pltpu.VMEM((BPT, D), jnp.float32),
            pltpu.VMEM((BPT, D), jnp.float32),
            pltpu.VMEM((BPT, D), jnp.float32),
            pltpu.VMEM((BPT, D), jnp.float32),
            pltpu.VMEM((BPT, D), jnp.float32),
            pltpu.VMEM((BPT, D), jnp.float32),
            pltpu.VMEM((BPT,), jnp.float32),
            pltpu.VMEM((BPT,), jnp.float32),
            pltpu.SemaphoreType.DMA,
        ],
    )()


def kernel(userId, itemId, neg_itemId, user_emb, item_emb, edge_row, edge_col, edge_weight):
    E = edge_row.shape[0]
    U = E // 2
    Upad = -(-U // 2048) * 2048
    pad = Upad - U

    def padhalf(x, fill):
        h0 = jnp.concatenate([x[:U], jnp.full((pad,), fill, x.dtype)])
        h1 = jnp.concatenate([x[U:], jnp.full((pad,), fill, x.dtype)])
        return jnp.concatenate([h0, h1])

    row_p = padhalf(edge_row, jnp.int32(1 << 30))
    col_p = padhalf(edge_col, jnp.int32(0))
    w_p = padhalf(edge_weight, jnp.float32(0.0))
    zeros = jnp.zeros((EB, D), jnp.float32)

    all_emb = jnp.concatenate([user_emb, item_emb], axis=0)
    spmm = _make_spmm(Upad)
    e1 = spmm(all_emb, row_p, col_p, w_p, zeros)
    e2 = spmm(e1, row_p, col_p, w_p, zeros)
    e3 = spmm(e2, row_p, col_p, w_p, zeros)

    final = _make_final()
    pos, neg, uego, pego, nego = final(
        all_emb, e1, e2, e3, user_emb, item_emb, userId, itemId, neg_itemId)
    return (pos, neg, uego, pego, nego)


# speedup vs baseline: 3.8297x; 1.0919x over previous
import functools

import jax
import jax.numpy as jnp
from jax import lax
from jax.experimental import pallas as pl
from jax.experimental.pallas import tpu as pltpu
from jax.experimental.pallas import tpu_sc as plsc

N_U = 50000
N_I = 50000
N = N_U + N_I
D = 64
CHUNK = 25000
ACC_ROWS = CHUNK + 8
EB = 128
ROWS_PT = 1560


def _spmm_body(Upad, in_hbm, row_hbm, col_hbm, w_hbm, zeros_hbm, out_hbm,
               acc, rows_v, col_v, row_v, w_v, idx_v, zbuf, sem):
    c = lax.axis_index("c")
    s = lax.axis_index("s")
    per_tile = Upad // 16
    nb = per_tile // EB

    pltpu.sync_copy(zeros_hbm, zbuf)

    for t in range(2):
        chunk_lo = t * N_U + c * CHUNK

        zstart = s * ROWS_PT
        for j in range(ROWS_PT // EB):
            pltpu.sync_copy(zbuf, acc.at[pl.ds(zstart + j * EB, EB)])
        rem = ROWS_PT - (ROWS_PT // EB) * EB
        if rem:
            pltpu.sync_copy(zbuf.at[pl.ds(0, rem)],
                            acc.at[pl.ds(zstart + (ROWS_PT // EB) * EB, rem)])

        @pl.when(s == 0)
        def _():
            pltpu.sync_copy(zbuf.at[pl.ds(0, 48)],
                            acc.at[pl.ds(16 * ROWS_PT, 48)])

        plsc.subcore_barrier()

        ebase0 = t * Upad + s * per_tile

        def batch_body(b, carry):
            ebase = ebase0 + b * EB
            pltpu.sync_copy(row_hbm.at[pl.ds(ebase, EB)], row_v)

            cvec = jnp.zeros((16,), jnp.int32)
            for i in range(EB // 16):
                rv = row_v[pl.ds(i * 16, 16)]
                local = rv - chunk_lo
                ok = (local >= 0) & (local < CHUNK)
                idx_v[pl.ds(i * 16, 16)] = jnp.where(ok, local, CHUNK)
                cvec = cvec + jnp.where(ok, 1, 0)
            cnt = cvec[0]
            for tt in range(1, 16):
                cnt = cnt + cvec[tt]

            @pl.when(cnt > 0)
            def _():
                pltpu.sync_copy(col_hbm.at[pl.ds(ebase, EB)], col_v)
                pltpu.sync_copy(w_hbm.at[pl.ds(ebase, EB)], w_v)
                pltpu.async_copy(in_hbm.at[col_v], rows_v, sem).wait()

                for i in range(EB // 16):
                    wv = w_v[pl.ds(i * 16, 16)]
                    for j in range(16):
                        e = i * 16 + j
                        ws = wv[j]
                        for q in range(D // 16):
                            rq = rows_v[e, pl.ds(q * 16, 16)]
                            rows_v[e, pl.ds(q * 16, 16)] = rq * ws

                pltpu.sync_copy(rows_v, acc.at[idx_v], add=True)

            return carry

        lax.fori_loop(0, nb, batch_body, 0)

        plsc.subcore_barrier()

        for j in range(ROWS_PT // EB):
            pltpu.sync_copy(acc.at[pl.ds(zstart + j * EB, EB)],
                            out_hbm.at[pl.ds(chunk_lo + zstart + j * EB, EB)])
        if rem:
            pltpu.sync_copy(
                acc.at[pl.ds(zstart + (ROWS_PT // EB) * EB, rem)],
                out_hbm.at[pl.ds(chunk_lo + zstart + (ROWS_PT // EB) * EB, rem)])

        @pl.when(s == 0)
        def _():
            pltpu.sync_copy(acc.at[pl.ds(16 * ROWS_PT, 40)],
                            out_hbm.at[pl.ds(chunk_lo + 16 * ROWS_PT, 40)])

        plsc.subcore_barrier()


def _make_spmm(Upad):
    mesh = plsc.VectorSubcoreMesh(core_axis_name="c", subcore_axis_name="s")
    return functools.partial(
        pl.kernel,
        functools.partial(_spmm_body, Upad),
        mesh=mesh,
        compiler_params=pltpu.CompilerParams(use_tc_tiling_on_sc=False),
        out_type=jax.ShapeDtypeStruct((N, D), jnp.float32),
        scratch_types=[
            pltpu.VMEM_SHARED((ACC_ROWS, D), jnp.float32),
            pltpu.VMEM((EB, D), jnp.float32),
            pltpu.VMEM((EB,), jnp.int32),
            pltpu.VMEM((EB,), jnp.int32),
            pltpu.VMEM((EB,), jnp.float32),
            pltpu.VMEM((EB,), jnp.int32),
            pltpu.VMEM((EB, D), jnp.float32),
            pltpu.SemaphoreType.DMA,
        ],
    )()


BPT = 128


def _final_body(e0, e1, e2, e3, ue_hbm, ie_hbm, uid_hbm, pid_hbm, nid_hbm,
                pos_hbm, neg_hbm, uego_hbm, pego_hbm, nego_hbm,
                uid_v, pid_v, nid_v, sid_v, b0, b1, b2, b3, fu, fp, fn,
                pos_v, neg_v, sem):
    c = lax.axis_index("c")
    s = lax.axis_index("s")
    wid = s * 2 + c
    base = wid * BPT

    pltpu.sync_copy(uid_hbm.at[pl.ds(base, BPT)], uid_v)
    pltpu.sync_copy(pid_hbm.at[pl.ds(base, BPT)], pid_v)
    pltpu.sync_copy(nid_hbm.at[pl.ds(base, BPT)], nid_v)

    pltpu.async_copy(ue_hbm.at[uid_v], b0, sem).wait()
    pltpu.sync_copy(b0, uego_hbm.at[pl.ds(base, BPT)])
    pltpu.async_copy(ie_hbm.at[pid_v], b0, sem).wait()
    pltpu.sync_copy(b0, pego_hbm.at[pl.ds(base, BPT)])
    pltpu.async_copy(ie_hbm.at[nid_v], b0, sem).wait()
    pltpu.sync_copy(b0, nego_hbm.at[pl.ds(base, BPT)])

    def combine(idv, dst):
        pltpu.async_copy(e0.at[idv], b0, sem).wait()
        pltpu.async_copy(e1.at[idv], b1, sem).wait()
        pltpu.async_copy(e2.at[idv], b2, sem).wait()
        pltpu.async_copy(e3.at[idv], b3, sem).wait()

        def row_body(r, carry):
            for q in range(D // 16):
                sl = pl.ds(q * 16, 16)
                v = (b0[r, sl] + b1[r, sl] + b2[r, sl] + b3[r, sl]) * 0.25
                dst[r, sl] = v
            return carry

        lax.fori_loop(0, BPT, row_body, 0)

    combine(uid_v, fu)

    for i in range(BPT // 16):
        sl = pl.ds(i * 16, 16)
        sid_v[sl] = pid_v[sl] + N_U
    combine(sid_v, fp)
    for i in range(BPT // 16):
        sl = pl.ds(i * 16, 16)
        sid_v[sl] = nid_v[sl] + N_U
    combine(sid_v, fn)

    lane = jax.lax.iota(jnp.int32, 16)
    for g in range(BPT // 16):
        def dot_body(j, carry):
            vp, vn = carry
            e = g * 16 + j
            accp = jnp.zeros((16,), jnp.float32)
            accn = jnp.zeros((16,), jnp.float32)
            for q in range(D // 16):
                sl = pl.ds(q * 16, 16)
                uvq = fu[e, sl]
                accp = accp + uvq * fp[e, sl]
                accn = accn + uvq * fn[e, sl]
            sp = accp[0]
            sn = accn[0]
            for t in range(1, 16):
                sp = sp + accp[t]
                sn = sn + accn[t]
            vp = jnp.where(lane == j, sp, vp)
            vn = jnp.where(lane == j, sn, vn)
            return vp, vn

        zp = jnp.zeros((16,), jnp.float32)
        vp, vn = lax.fori_loop(0, 16, dot_body, (zp, zp))
        pos_v[pl.ds(g * 16, 16)] = vp
        neg_v[pl.ds(g * 16, 16)] = vn

    pltpu.sync_copy(pos_v, pos_hbm.at[pl.ds(base, BPT)])
    pltpu.sync_copy(neg_v, neg_hbm.at[pl.ds(base, BPT)])


def _make_final():
    mesh = plsc.VectorSubcoreMesh(core_axis_name="c", subcore_axis_name="s")
    B = 4096
    return functools.partial(
        pl.kernel,
        _final_body,
        mesh=mesh,
        compiler_params=pltpu.CompilerParams(use_tc_tiling_on_sc=False),
        out_type=(
            jax.ShapeDtypeStruct((B,), jnp.float32),
            jax.ShapeDtypeStruct((B,), jnp.float32),
            jax.ShapeDtypeStruct((B, D), jnp.float32),
            jax.ShapeDtypeStruct((B, D), jnp.float32),
            jax.ShapeDtypeStruct((B, D), jnp.float32),
        ),
        scratch_types=[
            pltpu.VMEM((BPT,), jnp.int32),
            pltpu.VMEM((BPT,), jnp.int32),
            pltpu.VMEM((BPT,), jnp.int32),
            pltpu.VMEM((BPT,), jnp.int32),
            pltpu.VMEM((BPT, D), jnp.float32),
            pltpu.VMEM((BPT, D), jnp.float32),
            pltpu.VMEM((BPT, D), jnp.float32),
            pltpu.VMEM((BPT, D), jnp.float32),
            pltpu.VMEM((BPT, D), jnp.float32),
            pltpu.VMEM((BPT, D), jnp.float32),
            pltpu.VMEM((BPT, D), jnp.float32),
            pltpu.VMEM((BPT,), jnp.float32),
            pltpu.VMEM((BPT,), jnp.float32),
            pltpu.SemaphoreType.DMA,
        ],
    )()


def kernel(userId, itemId, neg_itemId, user_emb, item_emb, edge_row, edge_col, edge_weight):
    E = edge_row.shape[0]
    U = E // 2
    Upad = -(-U // 2048) * 2048
    pad = Upad - U

    def padhalf(x, fill):
        h0 = jnp.concatenate([x[:U], jnp.full((pad,), fill, x.dtype)])
        h1 = jnp.concatenate([x[U:], jnp.full((pad,), fill, x.dtype)])
        return jnp.concatenate([h0, h1])

    row_p = padhalf(edge_row, jnp.int32(1 << 30))
    col_p = padhalf(edge_col, jnp.int32(0))
    w_p = padhalf(edge_weight, jnp.float32(0.0))
    zeros = jnp.zeros((EB, D), jnp.float32)

    all_emb = jnp.concatenate([user_emb, item_emb], axis=0)
    spmm = _make_spmm(Upad)
    e1 = spmm(all_emb, row_p, col_p, w_p, zeros)
    e2 = spmm(e1, row_p, col_p, w_p, zeros)
    e3 = spmm(e2, row_p, col_p, w_p, zeros)

    final = _make_final()
    pos, neg, uego, pego, nego = final(
        all_emb, e1, e2, e3, user_emb, item_emb, userId, itemId, neg_itemId)
    return (pos, neg, uego, pego, nego)
